# Initial kernel scaffold; baseline (speedup 1.0000x reference)
import jax
import jax.numpy as jnp
from jax.experimental import pallas as pl


def kernel(h_id, r_id, edge_index, rel_id, ent_emb, rel_emb, neigh_w, gru_w_ih, gru_w_hh, gru_b_ih, gru_b_hh, gru_h0):
    raise NotImplementedError("write your pallas kernel here")


# speedup vs baseline: 4.8689x; 4.8689x over previous
import functools

import jax
import jax.numpy as jnp
from jax import lax
from jax.experimental import pallas as pl
from jax.experimental.pallas import tpu as pltpu
from jax.experimental.pallas import tpu_sc as plsc

N_ENT = 10000
N_PAD = 10240
N_REL2 = 402
H = 128
E = 320000
BS = 1024
NB = 134
NW = 32
EPW = E // NW
CHUNK = 80
NCHUNK = EPW // CHUNK
ROWS_PW = N_PAD // 16

_mesh = plsc.VectorSubcoreMesh(core_axis_name="c", subcore_axis_name="s")

_GDN = lax.GatherDimensionNumbers(
    offset_dims=(), collapsed_slice_dims=(0,), start_index_map=(0,))


def _rot(v, idx):
    return lax.gather(v, idx[:, None], dimension_numbers=_GDN,
                      slice_sizes=(1,),
                      mode=lax.GatherScatterMode.PROMISE_IN_BOUNDS)


def _rot_i(v, idx):
    return lax.gather(v, idx[:, None], dimension_numbers=_GDN,
                      slice_sizes=(1,),
                      mode=lax.GatherScatterMode.PROMISE_IN_BOUNDS)


def _splat_sum(v, lane):
    for k in (8, 4, 2, 1):
        v = v + _rot(v, jnp.bitwise_and(lane + k, 15))
    return v


@functools.partial(
    pl.kernel,
    mesh=_mesh,
    out_type=[
        jax.ShapeDtypeStruct((2, N_PAD, H), jnp.float32),
        jax.ShapeDtypeStruct((2, N_PAD // 128, 128), jnp.float32),
    ],
    scratch_types=[
        pltpu.VMEM_SHARED((N_PAD, H), jnp.float32),
        pltpu.VMEM_SHARED((N_PAD // 128, 128), jnp.float32),
        pltpu.VMEM((CHUNK,), jnp.int32),
        pltpu.VMEM((CHUNK,), jnp.int32),
        pltpu.VMEM((CHUNK,), jnp.int32),
        pltpu.VMEM((CHUNK, H), jnp.float32),
        pltpu.VMEM((CHUNK, H), jnp.float32),
        pltpu.VMEM((CHUNK, H), jnp.float32),
        pltpu.VMEM((CHUNK, 128), jnp.float32),
        pltpu.VMEM((CHUNK,), jnp.int32),
        pltpu.SemaphoreType.DMA,
    ],
)
def _edge_kernel(src_hbm, dst_hbm, rel_hbm, ent_hbm, relemb_hbm,
                 nu_hbm, den_hbm,
                 nacc, dacc, sidx, didx, ridx, srows, rrows, drows,
                 dbuf, didxr, sem):
    cid = lax.axis_index("c")
    sid = lax.axis_index("s")
    wid = sid * 2 + cid
    lane = lax.iota(jnp.int32, 16)

    zv = jnp.zeros((16,), jnp.float32)

    def _zrow(e, _):
        for hb in range(H // 16):
            srows[e, pl.ds(hb * 16, 16)] = zv
            dbuf[e, pl.ds(hb * 16, 16)] = zv
        return 0

    lax.fori_loop(0, CHUNK, _zrow, 0)

    @pl.when(sid == 0)
    def _():
        pltpu.sync_copy(dbuf, dacc)

    def _zcp(j, _):
        rs = pl.ds(sid * ROWS_PW + j * CHUNK, CHUNK)
        pltpu.sync_copy(srows, nacc.at[rs])
        return 0

    lax.fori_loop(0, ROWS_PW // CHUNK, _zcp, 0)
    plsc.subcore_barrier()

    def _chunk(k, _):
        base = wid * EPW + k * CHUNK
        pltpu.sync_copy(src_hbm.at[pl.ds(base, CHUNK)], sidx)
        pltpu.sync_copy(dst_hbm.at[pl.ds(base, CHUNK)], didx)
        pltpu.sync_copy(rel_hbm.at[pl.ds(base, CHUNK)], ridx)
        pltpu.async_copy(ent_hbm.at[sidx], srows, sem).wait()
        pltpu.async_copy(relemb_hbm.at[ridx], rrows, sem).wait()
        pltpu.async_copy(ent_hbm.at[didx], drows, sem).wait()

        def _edge(e, _):
            acc = jnp.zeros((16,), jnp.float32)
            for hb in range(H // 16):
                sl = pl.ds(hb * 16, 16)
                c = srows[e, sl] + rrows[e, sl]
                srows[e, sl] = c
                acc = acc + c * drows[e, sl]
            wv = jnp.exp(_splat_sum(acc, lane))
            for hb in range(H // 16):
                sl = pl.ds(hb * 16, 16)
                srows[e, sl] = srows[e, sl] * wv
            ej = jnp.bitwise_and(e, 15)
            dgv = didx[pl.ds(e - ej, 16)]
            d_vec = _rot_i(dgv, jnp.full((16,), ej, jnp.int32))
            tb = lax.shift_right_logical(jnp.bitwise_and(d_vec, 127), 4)
            tl = jnp.bitwise_and(d_vec, 15)
            lmask = 1.0 - jnp.minimum(
                jnp.bitwise_xor(lane, tl), 1).astype(jnp.float32)
            wrow = wv * lmask
            for blk in range(H // 16):
                bv = jnp.full((16,), blk, jnp.int32)
                bmask = 1.0 - jnp.minimum(
                    jnp.bitwise_xor(tb, bv), 1).astype(jnp.float32)
                dbuf[e, pl.ds(blk * 16, 16)] = wrow * bmask
            return 0

        lax.fori_loop(0, CHUNK, _edge, 0)

        def _dr(g, _):
            sl = pl.ds(g * 16, 16)
            didxr[sl] = lax.shift_right_logical(didx[sl], 7)
            return 0

        lax.fori_loop(0, CHUNK // 16, _dr, 0)

        pltpu.sync_copy(srows, nacc.at[didx], add=True)
        pltpu.sync_copy(dbuf, dacc.at[didxr], add=True)
        return 0

    lax.fori_loop(0, NCHUNK, _chunk, 0)
    plsc.subcore_barrier()

    def _rd(j, _):
        rs = pl.ds(sid * ROWS_PW + j * CHUNK, CHUNK)
        pltpu.sync_copy(nacc.at[rs], srows)
        pltpu.sync_copy(srows, nu_hbm.at[cid, rs])
        return 0

    lax.fori_loop(0, ROWS_PW // CHUNK, _rd, 0)

    @pl.when(sid == 0)
    def _():
        pltpu.sync_copy(dacc, den_hbm.at[cid])


@functools.partial(
    pl.kernel,
    mesh=_mesh,
    out_type=jax.ShapeDtypeStruct((BS, H), jnp.float32),
    scratch_types=[
        pltpu.VMEM((32,), jnp.int32),
        pltpu.VMEM((32,), jnp.int32),
        pltpu.VMEM((32, H), jnp.float32),
        pltpu.VMEM((32, H), jnp.float32),
        pltpu.SemaphoreType.DMA,
    ],
)
def _query_kernel(hid_hbm, rid_hbm, entout_hbm, relout_hbm, q_hbm,
                  hidx, ridx, hrows, rrows, sem):
    cid = lax.axis_index("c")
    sid = lax.axis_index("s")
    wid = sid * 2 + cid
    base = wid * (BS // NW)
    pltpu.sync_copy(hid_hbm.at[pl.ds(base, 32)], hidx)
    pltpu.sync_copy(rid_hbm.at[pl.ds(base, 32)], ridx)
    pltpu.async_copy(entout_hbm.at[hidx], hrows, sem).wait()
    pltpu.async_copy(relout_hbm.at[ridx], rrows, sem).wait()

    def _row(i, _):
        for hb in range(H // 16):
            sl = pl.ds(hb * 16, 16)
            hrows[i, sl] = hrows[i, sl] * rrows[i, sl]
        return 0

    lax.fori_loop(0, 32, _row, 0)
    pltpu.sync_copy(hrows, q_hbm.at[pl.ds(base, 32)])


def _entout_body(nu_ref, den_ref, w_ref, out_ref):
    nu = nu_ref[0] + nu_ref[1]
    d = den_ref[...]
    inv = jnp.where(d > 0.0, 1.0 / jnp.where(d > 0.0, d, 1.0), 0.0)
    rows = nu * inv
    out_ref[...] = jnp.tanh(
        lax.dot_general(rows, w_ref[...], (((1,), (0,)), ((), ())),
                        preferred_element_type=jnp.float32))


def _gru_body(xs_ref, wih_ref, whh_ref, bih_ref, bhh_ref, h0_ref, out_ref):
    h = h0_ref[...]
    wih = wih_ref[...]
    whh = whh_ref[...]
    bih = bih_ref[...]
    bhh = bhh_ref[...]
    for t in range(3):
        x = xs_ref[t]
        gi = lax.dot_general(x, wih, (((1,), (1,)), ((), ())),
                             preferred_element_type=jnp.float32) + bih
        gh = lax.dot_general(h, whh, (((1,), (1,)), ((), ())),
                             preferred_element_type=jnp.float32) + bhh
        r = 1.0 / (1.0 + jnp.exp(-(gi[:, 0:H] + gh[:, 0:H])))
        z = 1.0 / (1.0 + jnp.exp(-(gi[:, H:2 * H] + gh[:, H:2 * H])))
        n = jnp.tanh(gi[:, 2 * H:] + r * gh[:, 2 * H:])
        h = (1.0 - z) * n + z * h
        out_ref[t] = jnp.tanh(h)


def _score_body(q_ref, ent_ref, out_ref):
    s = lax.dot_general(q_ref[...], ent_ref[...], (((1,), (1,)), ((), ())),
                        preferred_element_type=jnp.float32)
    out_ref[...] = 1.0 / (1.0 + jnp.exp(-s))


def kernel(h_id, r_id, edge_index, rel_id, ent_emb, rel_emb, neigh_w,
           gru_w_ih, gru_w_hh, gru_b_ih, gru_b_hh, gru_h0):
    src = edge_index[0]
    dst = edge_index[1]
    nu, den = _edge_kernel(src.astype(jnp.int32), dst.astype(jnp.int32),
                           rel_id.astype(jnp.int32), ent_emb, rel_emb)
    den_vec = (den[0] + den[1]).reshape(N_PAD, 1)

    ent_out = pl.pallas_call(
        _entout_body,
        grid=(10,),
        in_specs=[
            pl.BlockSpec((2, N_PAD // 10, H), lambda i: (0, i, 0)),
            pl.BlockSpec((N_PAD // 10, 1), lambda i: (i, 0)),
            pl.BlockSpec((H, H), lambda i: (0, 0)),
        ],
        out_specs=pl.BlockSpec((N_PAD // 10, H), lambda i: (i, 0)),
        out_shape=jax.ShapeDtypeStruct((N_PAD, H), jnp.float32),
    )(nu, den_vec, neigh_w)

    xs = rel_emb[: 3 * NB, :].reshape(3, NB, H)
    rel_out = pl.pallas_call(
        _gru_body,
        out_shape=jax.ShapeDtypeStruct((3, NB, H), jnp.float32),
    )(xs, gru_w_ih, gru_w_hh, gru_b_ih.reshape(1, 3 * H),
      gru_b_hh.reshape(1, 3 * H), gru_h0).reshape(3 * NB, H)

    q = _query_kernel(h_id.astype(jnp.int32), r_id.astype(jnp.int32),
                      ent_out, rel_out)

    score = pl.pallas_call(
        _score_body,
        grid=(5,),
        in_specs=[
            pl.BlockSpec((BS, H), lambda i: (0, 0)),
            pl.BlockSpec((2048, H), lambda i: (i, 0)),
        ],
        out_specs=pl.BlockSpec((BS, 2048), lambda i: (0, i)),
        out_shape=jax.ShapeDtypeStruct((BS, N_ENT), jnp.float32),
    )(q, ent_out)
    return score


# speedup vs baseline: 5.8636x; 1.2043x over previous
import functools

import jax
import jax.numpy as jnp
from jax import lax
from jax.experimental import pallas as pl
from jax.experimental.pallas import tpu as pltpu
from jax.experimental.pallas import tpu_sc as plsc

N_ENT = 10000
N_PAD = 10240
N_REL2 = 402
H = 128
E = 320000
BS = 1024
NB = 134
NW = 32
EPW = E // NW
CHUNK = 80
NCHUNK = EPW // CHUNK
ROWS_PW = N_PAD // 16

_mesh = plsc.VectorSubcoreMesh(core_axis_name="c", subcore_axis_name="s")

_GDN = lax.GatherDimensionNumbers(
    offset_dims=(), collapsed_slice_dims=(0,), start_index_map=(0,))


def _rot(v, idx):
    return lax.gather(v, idx[:, None], dimension_numbers=_GDN,
                      slice_sizes=(1,),
                      mode=lax.GatherScatterMode.PROMISE_IN_BOUNDS)


def _rot_i(v, idx):
    return lax.gather(v, idx[:, None], dimension_numbers=_GDN,
                      slice_sizes=(1,),
                      mode=lax.GatherScatterMode.PROMISE_IN_BOUNDS)


def _splat_sum(v, lane):
    for k in (8, 4, 2, 1):
        v = v + _rot(v, jnp.bitwise_and(lane + k, 15))
    return v


@functools.partial(
    pl.kernel,
    mesh=_mesh,
    out_type=[
        jax.ShapeDtypeStruct((2, N_PAD, H), jnp.float32),
        jax.ShapeDtypeStruct((2, N_PAD // 128, 128), jnp.float32),
    ],
    scratch_types=[
        pltpu.VMEM_SHARED((N_PAD, H), jnp.float32),
        pltpu.VMEM_SHARED((N_PAD // 128, 128), jnp.float32),
        pltpu.VMEM((CHUNK,), jnp.int32),
        pltpu.VMEM((CHUNK,), jnp.int32),
        pltpu.VMEM((CHUNK,), jnp.int32),
        pltpu.VMEM((CHUNK, H), jnp.float32),
        pltpu.VMEM((CHUNK, H), jnp.float32),
        pltpu.VMEM((CHUNK, H), jnp.float32),
        pltpu.VMEM((CHUNK, 128), jnp.float32),
        pltpu.VMEM((CHUNK,), jnp.int32),
        pltpu.SemaphoreType.DMA,
    ],
)
def _edge_kernel(src_hbm, dst_hbm, rel_hbm, ent_hbm, relemb_hbm,
                 nu_hbm, den_hbm,
                 nacc, dacc, sidx, didx, ridx, srows, rrows, drows,
                 dbuf, didxr, sem):
    cid = lax.axis_index("c")
    sid = lax.axis_index("s")
    wid = sid * 2 + cid
    lane = lax.iota(jnp.int32, 16)

    zv = jnp.zeros((16,), jnp.float32)

    def _zrow(e, _):
        for hb in range(H // 16):
            srows[e, pl.ds(hb * 16, 16)] = zv
            dbuf[e, pl.ds(hb * 16, 16)] = zv
        return 0

    lax.fori_loop(0, CHUNK, _zrow, 0)

    @pl.when(sid == 0)
    def _():
        pltpu.sync_copy(dbuf, dacc)

    def _zcp(j, _):
        rs = pl.ds(sid * ROWS_PW + j * CHUNK, CHUNK)
        pltpu.sync_copy(srows, nacc.at[rs])
        return 0

    lax.fori_loop(0, ROWS_PW // CHUNK, _zcp, 0)
    plsc.subcore_barrier()

    def _chunk(k, _):
        base = wid * EPW + k * CHUNK
        i1 = pltpu.async_copy(src_hbm.at[pl.ds(base, CHUNK)], sidx, sem)
        i2 = pltpu.async_copy(dst_hbm.at[pl.ds(base, CHUNK)], didx, sem)
        i3 = pltpu.async_copy(rel_hbm.at[pl.ds(base, CHUNK)], ridx, sem)
        i1.wait()
        i2.wait()
        i3.wait()
        g1 = pltpu.async_copy(ent_hbm.at[sidx], srows, sem)
        g2 = pltpu.async_copy(relemb_hbm.at[ridx], rrows, sem)
        g3 = pltpu.async_copy(ent_hbm.at[didx], drows, sem)
        g1.wait()
        g2.wait()
        g3.wait()

        def _edge(e, _):
            acc = jnp.zeros((16,), jnp.float32)
            for hb in range(H // 16):
                sl = pl.ds(hb * 16, 16)
                c = srows[e, sl] + rrows[e, sl]
                srows[e, sl] = c
                acc = acc + c * drows[e, sl]
            wv = jnp.exp(_splat_sum(acc, lane))
            for hb in range(H // 16):
                sl = pl.ds(hb * 16, 16)
                srows[e, sl] = srows[e, sl] * wv
            ej = jnp.bitwise_and(e, 15)
            dgv = didx[pl.ds(e - ej, 16)]
            d_vec = _rot_i(dgv, jnp.full((16,), ej, jnp.int32))
            tb = lax.shift_right_logical(jnp.bitwise_and(d_vec, 127), 4)
            tl = jnp.bitwise_and(d_vec, 15)
            lmask = 1.0 - jnp.minimum(
                jnp.bitwise_xor(lane, tl), 1).astype(jnp.float32)
            wrow = wv * lmask
            for blk in range(H // 16):
                bv = jnp.full((16,), blk, jnp.int32)
                bmask = 1.0 - jnp.minimum(
                    jnp.bitwise_xor(tb, bv), 1).astype(jnp.float32)
                dbuf[e, pl.ds(blk * 16, 16)] = wrow * bmask
            return 0

        lax.fori_loop(0, CHUNK, _edge, 0)

        def _dr(g, _):
            sl = pl.ds(g * 16, 16)
            didxr[sl] = lax.shift_right_logical(didx[sl], 7)
            return 0

        lax.fori_loop(0, CHUNK // 16, _dr, 0)

        s1 = pltpu.async_copy(srows, nacc.at[didx], sem, add=True)
        s2 = pltpu.async_copy(dbuf, dacc.at[didxr], sem, add=True)
        s1.wait()
        s2.wait()
        return 0

    lax.fori_loop(0, NCHUNK, _chunk, 0)
    plsc.subcore_barrier()

    def _rd(j, _):
        rs = pl.ds(sid * ROWS_PW + j * CHUNK, CHUNK)
        pltpu.sync_copy(nacc.at[rs], srows)
        pltpu.sync_copy(srows, nu_hbm.at[cid, rs])
        return 0

    lax.fori_loop(0, ROWS_PW // CHUNK, _rd, 0)

    @pl.when(sid == 0)
    def _():
        pltpu.sync_copy(dacc, den_hbm.at[cid])


@functools.partial(
    pl.kernel,
    mesh=_mesh,
    out_type=jax.ShapeDtypeStruct((BS, H), jnp.float32),
    scratch_types=[
        pltpu.VMEM((32,), jnp.int32),
        pltpu.VMEM((32,), jnp.int32),
        pltpu.VMEM((32, H), jnp.float32),
        pltpu.VMEM((32, H), jnp.float32),
        pltpu.SemaphoreType.DMA,
    ],
)
def _query_kernel(hid_hbm, rid_hbm, entout_hbm, relout_hbm, q_hbm,
                  hidx, ridx, hrows, rrows, sem):
    cid = lax.axis_index("c")
    sid = lax.axis_index("s")
    wid = sid * 2 + cid
    base = wid * (BS // NW)
    pltpu.sync_copy(hid_hbm.at[pl.ds(base, 32)], hidx)
    pltpu.sync_copy(rid_hbm.at[pl.ds(base, 32)], ridx)
    pltpu.async_copy(entout_hbm.at[hidx], hrows, sem).wait()
    pltpu.async_copy(relout_hbm.at[ridx], rrows, sem).wait()

    def _row(i, _):
        for hb in range(H // 16):
            sl = pl.ds(hb * 16, 16)
            hrows[i, sl] = hrows[i, sl] * rrows[i, sl]
        return 0

    lax.fori_loop(0, 32, _row, 0)
    pltpu.sync_copy(hrows, q_hbm.at[pl.ds(base, 32)])


def _entout_body(nu_ref, den_ref, w_ref, out_ref):
    nu = nu_ref[0] + nu_ref[1]
    d = den_ref[...]
    inv = jnp.where(d > 0.0, 1.0 / jnp.where(d > 0.0, d, 1.0), 0.0)
    rows = nu * inv
    out_ref[...] = jnp.tanh(
        lax.dot_general(rows, w_ref[...], (((1,), (0,)), ((), ())),
                        preferred_element_type=jnp.float32))


def _gru_body(xs_ref, wih_ref, whh_ref, bih_ref, bhh_ref, h0_ref, out_ref):
    h = h0_ref[...]
    wih = wih_ref[...]
    whh = whh_ref[...]
    bih = bih_ref[...]
    bhh = bhh_ref[...]
    for t in range(3):
        x = xs_ref[t]
        gi = lax.dot_general(x, wih, (((1,), (1,)), ((), ())),
                             preferred_element_type=jnp.float32) + bih
        gh = lax.dot_general(h, whh, (((1,), (1,)), ((), ())),
                             preferred_element_type=jnp.float32) + bhh
        r = 1.0 / (1.0 + jnp.exp(-(gi[:, 0:H] + gh[:, 0:H])))
        z = 1.0 / (1.0 + jnp.exp(-(gi[:, H:2 * H] + gh[:, H:2 * H])))
        n = jnp.tanh(gi[:, 2 * H:] + r * gh[:, 2 * H:])
        h = (1.0 - z) * n + z * h
        out_ref[t] = jnp.tanh(h)


def _score_body(q_ref, ent_ref, out_ref):
    s = lax.dot_general(q_ref[...], ent_ref[...], (((1,), (1,)), ((), ())),
                        preferred_element_type=jnp.float32)
    out_ref[...] = 1.0 / (1.0 + jnp.exp(-s))


def kernel(h_id, r_id, edge_index, rel_id, ent_emb, rel_emb, neigh_w,
           gru_w_ih, gru_w_hh, gru_b_ih, gru_b_hh, gru_h0):
    src = edge_index[0]
    dst = edge_index[1]
    nu, den = _edge_kernel(src.astype(jnp.int32), dst.astype(jnp.int32),
                           rel_id.astype(jnp.int32), ent_emb, rel_emb)
    den_vec = (den[0] + den[1]).reshape(N_PAD, 1)

    ent_out = pl.pallas_call(
        _entout_body,
        grid=(10,),
        in_specs=[
            pl.BlockSpec((2, N_PAD // 10, H), lambda i: (0, i, 0)),
            pl.BlockSpec((N_PAD // 10, 1), lambda i: (i, 0)),
            pl.BlockSpec((H, H), lambda i: (0, 0)),
        ],
        out_specs=pl.BlockSpec((N_PAD // 10, H), lambda i: (i, 0)),
        out_shape=jax.ShapeDtypeStruct((N_PAD, H), jnp.float32),
    )(nu, den_vec, neigh_w)

    xs = rel_emb[: 3 * NB, :].reshape(3, NB, H)
    rel_out = pl.pallas_call(
        _gru_body,
        out_shape=jax.ShapeDtypeStruct((3, NB, H), jnp.float32),
    )(xs, gru_w_ih, gru_w_hh, gru_b_ih.reshape(1, 3 * H),
      gru_b_hh.reshape(1, 3 * H), gru_h0).reshape(3 * NB, H)

    q = _query_kernel(h_id.astype(jnp.int32), r_id.astype(jnp.int32),
                      ent_out, rel_out)

    score = pl.pallas_call(
        _score_body,
        grid=(5,),
        in_specs=[
            pl.BlockSpec((BS, H), lambda i: (0, 0)),
            pl.BlockSpec((2048, H), lambda i: (i, 0)),
        ],
        out_specs=pl.BlockSpec((BS, 2048), lambda i: (0, i)),
        out_shape=jax.ShapeDtypeStruct((BS, N_ENT), jnp.float32),
    )(q, ent_out)
    return score
